# baseline (speedup 1.0000x reference)
import jax
import jax.numpy as jnp
from jax.experimental import pallas as pl
from jax.experimental.pallas import tpu as pltpu

_N = 6400000
_STRIPS = 8
_CHUNK = _N // _STRIPS


def _copy_body(in_ref, out_ref, sem):
    for i in range(_STRIPS):
        pltpu.make_async_copy(
            in_ref.at[pl.ds(i * _CHUNK, _CHUNK)],
            out_ref.at[pl.ds(i * _CHUNK, _CHUNK)],
            sem.at[i],
        ).start()
    for i in range(_STRIPS):
        pltpu.make_async_copy(
            in_ref.at[pl.ds(i * _CHUNK, _CHUNK)],
            out_ref.at[pl.ds(i * _CHUNK, _CHUNK)],
            sem.at[i],
        ).wait()


def kernel(Zj):
    return pl.pallas_call(
        _copy_body,
        out_shape=jax.ShapeDtypeStruct(Zj.shape, Zj.dtype),
        in_specs=[pl.BlockSpec(memory_space=pl.ANY)],
        out_specs=pl.BlockSpec(memory_space=pl.ANY),
        scratch_shapes=[pltpu.SemaphoreType.DMA((_STRIPS,))],
    )(Zj)


# speedup vs baseline: 29.9524x; 29.9524x over previous
import jax
import jax.numpy as jnp
from jax.experimental import pallas as pl
from jax.experimental.pallas import tpu as pltpu

_N = 6400000
_ROWS = 50000
_LANES = 128
_BLOCK_ROWS = 2000


def _copy_body(in_ref, out_ref):
    out_ref[...] = in_ref[...]


def kernel(Zj):
    x = Zj.reshape(_ROWS, _LANES)
    y = pl.pallas_call(
        _copy_body,
        out_shape=jax.ShapeDtypeStruct((_ROWS, _LANES), Zj.dtype),
        grid=(_ROWS // _BLOCK_ROWS,),
        in_specs=[pl.BlockSpec((_BLOCK_ROWS, _LANES), lambda i: (i, 0))],
        out_specs=pl.BlockSpec((_BLOCK_ROWS, _LANES), lambda i: (i, 0)),
    )(x)
    return y.reshape(_N)


# speedup vs baseline: 45.4047x; 1.5159x over previous
import jax
import jax.numpy as jnp
from jax.experimental import pallas as pl
from jax.experimental.pallas import tpu as pltpu

_N = 6400000
_ROWS = 50000
_LANES = 128
_BLOCK_ROWS = 10000


def _copy_body(in_ref, out_ref):
    out_ref[...] = in_ref[...]


def kernel(Zj):
    x = Zj.reshape(_ROWS, _LANES)
    y = pl.pallas_call(
        _copy_body,
        out_shape=jax.ShapeDtypeStruct((_ROWS, _LANES), Zj.dtype),
        grid=(_ROWS // _BLOCK_ROWS,),
        in_specs=[pl.BlockSpec((_BLOCK_ROWS, _LANES), lambda i: (i, 0))],
        out_specs=pl.BlockSpec((_BLOCK_ROWS, _LANES), lambda i: (i, 0)),
    )(x)
    return y.reshape(_N)


# speedup vs baseline: 49.0195x; 1.0796x over previous
import jax
import jax.numpy as jnp
from jax.experimental import pallas as pl
from jax.experimental.pallas import tpu as pltpu

_N = 6400000
_ROWS = 50000
_LANES = 128
_BLOCK_ROWS = 25000


def _copy_body(in_ref, out_ref):
    out_ref[...] = in_ref[...]


def kernel(Zj):
    x = Zj.reshape(_ROWS, _LANES)
    y = pl.pallas_call(
        _copy_body,
        out_shape=jax.ShapeDtypeStruct((_ROWS, _LANES), Zj.dtype),
        grid=(_ROWS // _BLOCK_ROWS,),
        in_specs=[pl.BlockSpec((_BLOCK_ROWS, _LANES), lambda i: (i, 0))],
        out_specs=pl.BlockSpec((_BLOCK_ROWS, _LANES), lambda i: (i, 0)),
    )(x)
    return y.reshape(_N)
